# baseline (speedup 1.0000x reference)
import functools

import jax
import jax.numpy as jnp
from jax import lax
from jax.experimental import pallas as pl
from jax.experimental.pallas import tpu as pltpu
from jax.experimental.pallas import tpu_sc as plsc

B, T, D = 2, 4096, 1024
S = T // 2
R = 1024
TBL = T + R

_PREC = lax.Precision.DEFAULT


def _scores_select_body(xr_ref, ge_ref, go_ref, dsel_ref, wsel_ref,
                        best_scr, bidx_scr, rank_ref):
    xs = xr_ref[0]
    dstm = xs[:, :D]
    srcm = xs[:, D:]
    dn = dstm / jnp.maximum(
        jnp.sqrt(jnp.sum(dstm * dstm, axis=1, keepdims=True)), 1e-12)
    sn = srcm / jnp.maximum(
        jnp.sqrt(jnp.sum(srcm * srcm, axis=1, keepdims=True)), 1e-12)

    CH = 256
    for c in range(S // CH):
        sc = lax.dot_general(sn[c * CH:(c + 1) * CH], dn,
                             (((1,), (1,)), ((), ())),
                             precision=_PREC,
                             preferred_element_type=jnp.float32)
        m = jnp.max(sc, axis=1)
        ii = lax.broadcasted_iota(jnp.int32, (CH, S), 1)
        am = jnp.min(jnp.where(sc == m[:, None], ii, S), axis=1)
        best_scr[c * CH:(c + 1) * CH] = m
        bidx_scr[c * CH:(c + 1) * CH] = am

    _select_work(best_scr[...], bidx_scr[...],
                 ge_ref, go_ref, dsel_ref, wsel_ref, rank_ref)


def _scores_select_call(xr):
    return pl.pallas_call(
        _scores_select_body,
        grid=(B,),
        in_specs=[pl.BlockSpec((1, S, 2 * D), lambda b: (b, 0, 0))],
        out_specs=[pl.BlockSpec((1, 1, S), lambda b: (b, 0, 0)),
                   pl.BlockSpec((1, 1, S), lambda b: (b, 0, 0)),
                   pl.BlockSpec((1, 1, R), lambda b: (b, 0, 0)),
                   pl.BlockSpec((1, 1, R), lambda b: (b, 0, 0))],
        out_shape=[jax.ShapeDtypeStruct((B, 1, S), jnp.int32),
                   jax.ShapeDtypeStruct((B, 1, S), jnp.int32),
                   jax.ShapeDtypeStruct((B, 1, R), jnp.int32),
                   jax.ShapeDtypeStruct((B, 1, R), jnp.int32)],
        scratch_shapes=[pltpu.VMEM((S,), jnp.float32),
                        pltpu.VMEM((S,), jnp.int32),
                        pltpu.VMEM((S,), jnp.float32)],
    )(xr)


def _select_work(sv, bi, ge_ref, go_ref, dsel_ref, wsel_ref, rank_ref):
    bif = bi.astype(jnp.float32)
    sv_row = sv[None, :]
    iota_row = lax.broadcasted_iota(jnp.int32, (1, S), 1)
    iota_row_f = iota_row.astype(jnp.float32)
    d_row_f = iota_row_f

    CH = 256
    NCH = S // CH
    BIG = jnp.float32(3.0)
    Rf = jnp.float32(R)
    ones_col = jnp.ones((S, 128), jnp.float32)

    wmin = jnp.full((S,), BIG, jnp.float32)
    for c in range(NCH):
        sl = slice(c * CH, (c + 1) * CH)
        svc = jnp.broadcast_to(sv[sl][:, None], (CH, S))
        idxc = lax.broadcasted_iota(jnp.int32, (CH, S), 0) + c * CH
        ind = jnp.where((sv_row > svc) |
                        ((sv_row == svc) & (iota_row < idxc)), 1.0, 0.0)
        rk = lax.dot_general(ind, ones_col, (((1,), (0,)), ((), ())),
                             precision=lax.Precision.DEFAULT,
                             preferred_element_type=jnp.float32)
        rkc = rk[:, 0:1]
        rank_ref[sl] = rkc[:, 0]
        bisel = jnp.broadcast_to(
            jnp.where(rkc < Rf, bif[sl][:, None], -1.0), (CH, S))
        hit = bisel == d_row_f
        wmin = jnp.minimum(wmin, jnp.min(jnp.where(hit, svc, BIG), axis=0))

    widx = jnp.full((S,), -1.0, jnp.float32)
    smin = jnp.full((S,), Rf, jnp.float32)
    for c in range(NCH):
        sl = slice(c * CH, (c + 1) * CH)
        rkc = rank_ref[sl][:, None]
        bisel = jnp.broadcast_to(
            jnp.where(rkc < Rf, bif[sl][:, None], -1.0), (CH, S))
        svc = jnp.broadcast_to(sv[sl][:, None], (CH, S))
        rkb = jnp.broadcast_to(rkc, (CH, S))
        idxcf = (lax.broadcasted_iota(jnp.int32, (CH, 1), 0)
                 .astype(jnp.float32) + c * CH)
        hit = bisel == d_row_f
        win = hit & (svc == wmin[None, :])
        widx = jnp.maximum(widx, jnp.max(jnp.where(win, idxcf, -1.0), axis=0))
        smin = jnp.minimum(smin, jnp.min(jnp.where(hit, rkb, Rf), axis=0))
    merged = widx >= 0.0

    slot_row = lax.broadcasted_iota(jnp.int32, (1, R), 1).astype(jnp.float32)
    bi_row = bi[None, :]
    sel_row = rank_ref[...][None, :] < Rf
    dsel = jnp.zeros((R,), jnp.int32)
    wsel = jnp.zeros((R,), jnp.float32)
    gov = jnp.zeros((S,), jnp.float32)
    for c in range(NCH):
        sl = slice(c * CH, (c + 1) * CH)
        widxc1 = widx[sl][:, None]
        smsel = jnp.broadcast_to(
            jnp.where(widxc1 >= 0.0, smin[sl][:, None], Rf + 1.0), (CH, R))
        widxc = jnp.broadcast_to(widxc1, (CH, R))
        dcf = (lax.broadcasted_iota(jnp.int32, (CH, R), 0) + c * CH)
        hitd = smsel == slot_row
        dsel = dsel + jnp.sum(jnp.where(hitd, 2 * dcf, 0), axis=0)
        wsel = wsel + jnp.sum(jnp.where(hitd, 2.0 * widxc + 1.0, 0.0),
                              axis=0)
        dcf2 = lax.broadcasted_iota(jnp.int32, (CH, S), 0) + c * CH
        sminc2 = jnp.broadcast_to(smin[sl][:, None], (CH, S))
        hit2 = (bi_row == dcf2) & sel_row
        gov = gov + jnp.sum(jnp.where(hit2, sminc2, 0.0), axis=0)

    dd = iota_row[0]
    ge = jnp.where(merged, T + smin.astype(jnp.int32), 2 * dd)
    go = jnp.where(rank_ref[...] < Rf, T + gov.astype(jnp.int32), 2 * dd + 1)

    ge_ref[0, 0, :] = ge
    go_ref[0, 0, :] = go
    dsel_ref[0, 0, :] = dsel
    wsel_ref[0, 0, :] = wsel.astype(jnp.int32)


_HBLK = 256
_NH = T // _HBLK
_NM = R // _HBLK


def _hidden_merge_body(x_ref, w_ref, b_ref, dsel_ref, wsel_ref, tbl_ref,
                       h_scr):
    j = pl.program_id(1)

    @pl.when(j < _NH)
    def _():
        hb = (lax.dot_general(x_ref[0], w_ref[...],
                              (((1,), (0,)), ((), ())),
                              precision=_PREC,
                              preferred_element_type=jnp.float32)
              + b_ref[...])
        tbl_ref[0] = hb
        h_scr[pl.ds(j * _HBLK, _HBLK), :] = hb

    @pl.when(j >= _NH)
    def _():
        ds_ = dsel_ref[0, 0, :][:, None]
        ws_ = wsel_ref[0, 0, :][:, None]
        tt = lax.broadcasted_iota(jnp.int32, (_HBLK, T), 1)
        E = (0.5 * (tt == ds_).astype(jnp.float32)
             + 0.5 * (tt == ws_).astype(jnp.float32))
        tbl_ref[0] = lax.dot_general(E, h_scr[...],
                                     (((1,), (0,)), ((), ())),
                                     precision=_PREC,
                                     preferred_element_type=jnp.float32)


def _hidden_merge_call(x3, W, b2d, dsel, wsel):
    nj = _NH + _NM
    return pl.pallas_call(
        _hidden_merge_body,
        grid=(B, nj),
        in_specs=[pl.BlockSpec((1, _HBLK, D),
                               lambda b, j: (b, jnp.minimum(j, _NH - 1), 0)),
                  pl.BlockSpec((D, D), lambda b, j: (0, 0)),
                  pl.BlockSpec((1, D), lambda b, j: (0, 0)),
                  pl.BlockSpec((1, 1, _HBLK),
                               lambda b, j: (b, 0, jnp.maximum(j - _NH, 0))),
                  pl.BlockSpec((1, 1, _HBLK),
                               lambda b, j: (b, 0, jnp.maximum(j - _NH, 0)))],
        out_specs=pl.BlockSpec((1, _HBLK, D), lambda b, j: (b, j, 0)),
        out_shape=jax.ShapeDtypeStruct((B, TBL, D), jnp.float32),
        scratch_shapes=[pltpu.VMEM((T, D), jnp.float32)],
    )(x3, W, b2d, dsel, wsel)


_NW = 32
_GCH = 32


def _gather_call(table, gidx):
    nrows = B * T
    per_w = nrows // _NW
    nchk = per_w // _GCH
    mesh = plsc.VectorSubcoreMesh(core_axis_name="c", subcore_axis_name="s")

    @functools.partial(
        pl.kernel,
        out_type=jax.ShapeDtypeStruct((nrows, D), jnp.float32),
        mesh=mesh,
        scratch_types=[pltpu.VMEM((_GCH,), jnp.int32),
                       pltpu.VMEM((_GCH,), jnp.int32),
                       pltpu.VMEM((_GCH, D), jnp.float32),
                       pltpu.VMEM((_GCH, D), jnp.float32),
                       pltpu.SemaphoreType.DMA,
                       pltpu.SemaphoreType.DMA,
                       pltpu.SemaphoreType.DMA,
                       pltpu.SemaphoreType.DMA],
    )
    def k(table_hbm, idx_hbm, out_hbm, iv0, iv1, rv0, rv1, gs0, gs1,
          ws0, ws1):
        wid = lax.axis_index("s") * 2 + lax.axis_index("c")
        base = wid * per_w
        ivs, rvs = (iv0, iv1), (rv0, rv1)
        gss, wss = (gs0, gs1), (ws0, ws1)

        @pl.loop(0, nchk, step=2)
        def _(k0):
            for bb in range(2):
                kk = k0 + bb
                off = base + kk * _GCH

                @pl.when(kk >= 2)
                def _():
                    pltpu.make_async_copy(table_hbm.at[pl.ds(0, _GCH)],
                                          rvs[bb], wss[bb]).wait()

                pltpu.sync_copy(idx_hbm.at[pl.ds(off, _GCH)], ivs[bb])
                pltpu.async_copy(table_hbm.at[ivs[bb]], rvs[bb],
                                 gss[bb]).wait()
                pltpu.async_copy(rvs[bb], out_hbm.at[pl.ds(off, _GCH)],
                                 wss[bb])

        for bb in range(2):
            pltpu.make_async_copy(table_hbm.at[pl.ds(0, _GCH)],
                                  rvs[bb], wss[bb]).wait()

    return k(table, gidx)


def kernel(x, W, b):
    xr = x.reshape(B, S, 2 * D)
    ge, go, dsel, wsel = _scores_select_call(xr)

    table = _hidden_merge_call(x, W, b.reshape(1, D), dsel, wsel)

    g = jnp.stack([ge[:, 0, :], go[:, 0, :]], axis=-1).reshape(B, T)
    g = g + (jnp.arange(B, dtype=jnp.int32) * TBL)[:, None]
    out = _gather_call(table.reshape(B * TBL, D), g.reshape(B * T))
    return out.reshape(B, T, D)


# speedup vs baseline: 1.0737x; 1.0737x over previous
import functools

import jax
import jax.numpy as jnp
from jax import lax
from jax.experimental import pallas as pl
from jax.experimental.pallas import tpu as pltpu
from jax.experimental.pallas import tpu_sc as plsc

B, T, D = 2, 4096, 1024
S = T // 2
R = 1024
TBL = T + R

_PREC = lax.Precision.DEFAULT


def _scores_select_body(xr_ref, ge_ref, go_ref, dsel_ref, wsel_ref,
                        best_scr, bidx_scr, rank_ref):
    xs = xr_ref[0]
    dstm = xs[:, :D]
    srcm = xs[:, D:]
    dn = dstm / jnp.maximum(
        jnp.sqrt(jnp.sum(dstm * dstm, axis=1, keepdims=True)), 1e-12)
    sn = srcm / jnp.maximum(
        jnp.sqrt(jnp.sum(srcm * srcm, axis=1, keepdims=True)), 1e-12)

    CH = 256
    for c in range(S // CH):
        sc = lax.dot_general(sn[c * CH:(c + 1) * CH], dn,
                             (((1,), (1,)), ((), ())),
                             precision=_PREC,
                             preferred_element_type=jnp.float32)
        m = jnp.max(sc, axis=1)
        ii = lax.broadcasted_iota(jnp.int32, (CH, S), 1)
        am = jnp.min(jnp.where(sc == m[:, None], ii, S), axis=1)
        best_scr[c * CH:(c + 1) * CH] = m
        bidx_scr[c * CH:(c + 1) * CH] = am

    _select_work(best_scr[...], bidx_scr[...],
                 ge_ref, go_ref, dsel_ref, wsel_ref, rank_ref)


def _scores_select_call(xr):
    return pl.pallas_call(
        _scores_select_body,
        grid=(B,),
        in_specs=[pl.BlockSpec((1, S, 2 * D), lambda b: (b, 0, 0))],
        out_specs=[pl.BlockSpec((1, 1, S), lambda b: (b, 0, 0)),
                   pl.BlockSpec((1, 1, S), lambda b: (b, 0, 0)),
                   pl.BlockSpec((1, 1, R), lambda b: (b, 0, 0)),
                   pl.BlockSpec((1, 1, R), lambda b: (b, 0, 0))],
        out_shape=[jax.ShapeDtypeStruct((B, 1, S), jnp.int32),
                   jax.ShapeDtypeStruct((B, 1, S), jnp.int32),
                   jax.ShapeDtypeStruct((B, 1, R), jnp.int32),
                   jax.ShapeDtypeStruct((B, 1, R), jnp.int32)],
        scratch_shapes=[pltpu.VMEM((S,), jnp.float32),
                        pltpu.VMEM((S,), jnp.int32),
                        pltpu.VMEM((S,), jnp.float32)],
    )(xr)


def _select_work(sv, bi, ge_ref, go_ref, dsel_ref, wsel_ref, rank_ref):
    bif = bi.astype(jnp.float32)
    sv_row = sv[None, :]
    iota_row = lax.broadcasted_iota(jnp.int32, (1, S), 1)
    iota_row_f = iota_row.astype(jnp.float32)
    d_row_f = iota_row_f

    CH = 256
    NCH = S // CH
    BIG = jnp.float32(3.0)
    Rf = jnp.float32(R)
    ones_col = jnp.ones((S, 128), jnp.float32)

    wmin = jnp.full((S,), BIG, jnp.float32)
    for c in range(NCH):
        sl = slice(c * CH, (c + 1) * CH)
        svc = jnp.broadcast_to(sv[sl][:, None], (CH, S))
        idxc = lax.broadcasted_iota(jnp.int32, (CH, S), 0) + c * CH
        ind = jnp.where((sv_row > svc) |
                        ((sv_row == svc) & (iota_row < idxc)), 1.0, 0.0)
        rk = lax.dot_general(ind, ones_col, (((1,), (0,)), ((), ())),
                             precision=lax.Precision.DEFAULT,
                             preferred_element_type=jnp.float32)
        rkc = rk[:, 0:1]
        rank_ref[sl] = rkc[:, 0]
        bisel = jnp.broadcast_to(
            jnp.where(rkc < Rf, bif[sl][:, None], -1.0), (CH, S))
        hit = bisel == d_row_f
        wmin = jnp.minimum(wmin, jnp.min(jnp.where(hit, svc, BIG), axis=0))

    widx = jnp.full((S,), -1.0, jnp.float32)
    smin = jnp.full((S,), Rf, jnp.float32)
    for c in range(NCH):
        sl = slice(c * CH, (c + 1) * CH)
        rkc = rank_ref[sl][:, None]
        bisel = jnp.broadcast_to(
            jnp.where(rkc < Rf, bif[sl][:, None], -1.0), (CH, S))
        svc = jnp.broadcast_to(sv[sl][:, None], (CH, S))
        rkb = jnp.broadcast_to(rkc, (CH, S))
        idxcf = (lax.broadcasted_iota(jnp.int32, (CH, 1), 0)
                 .astype(jnp.float32) + c * CH)
        hit = bisel == d_row_f
        win = hit & (svc == wmin[None, :])
        widx = jnp.maximum(widx, jnp.max(jnp.where(win, idxcf, -1.0), axis=0))
        smin = jnp.minimum(smin, jnp.min(jnp.where(hit, rkb, Rf), axis=0))
    merged = widx >= 0.0

    slot_row = lax.broadcasted_iota(jnp.int32, (1, R), 1).astype(jnp.float32)
    bi_row = bi[None, :]
    sel_row = rank_ref[...][None, :] < Rf
    dsel = jnp.zeros((R,), jnp.int32)
    wsel = jnp.zeros((R,), jnp.float32)
    gov = jnp.zeros((S,), jnp.float32)
    for c in range(NCH):
        sl = slice(c * CH, (c + 1) * CH)
        widxc1 = widx[sl][:, None]
        smsel = jnp.broadcast_to(
            jnp.where(widxc1 >= 0.0, smin[sl][:, None], Rf + 1.0), (CH, R))
        widxc = jnp.broadcast_to(widxc1, (CH, R))
        dcf = (lax.broadcasted_iota(jnp.int32, (CH, R), 0) + c * CH)
        hitd = smsel == slot_row
        dsel = dsel + jnp.sum(jnp.where(hitd, 2 * dcf, 0), axis=0)
        wsel = wsel + jnp.sum(jnp.where(hitd, 2.0 * widxc + 1.0, 0.0),
                              axis=0)
        dcf2 = lax.broadcasted_iota(jnp.int32, (CH, S), 0) + c * CH
        sminc2 = jnp.broadcast_to(smin[sl][:, None], (CH, S))
        hit2 = (bi_row == dcf2) & sel_row
        gov = gov + jnp.sum(jnp.where(hit2, sminc2, 0.0), axis=0)

    dd = iota_row[0]
    ge = jnp.where(merged, T + smin.astype(jnp.int32), 2 * dd)
    go = jnp.where(rank_ref[...] < Rf, T + gov.astype(jnp.int32), 2 * dd + 1)

    ge_ref[0, 0, :] = ge
    go_ref[0, 0, :] = go
    dsel_ref[0, 0, :] = dsel
    wsel_ref[0, 0, :] = wsel.astype(jnp.int32)


_HBLK = 512
_NH = T // _HBLK
_NM = R // _HBLK


def _hidden_merge_body(x_ref, w_ref, b_ref, dsel_ref, wsel_ref, tbl_ref,
                       h_scr):
    j = pl.program_id(1)

    @pl.when(j < _NH)
    def _():
        hb = (lax.dot_general(x_ref[0], w_ref[...],
                              (((1,), (0,)), ((), ())),
                              precision=_PREC,
                              preferred_element_type=jnp.float32)
              + b_ref[...])
        tbl_ref[0] = hb
        h_scr[pl.ds(j * _HBLK, _HBLK), :] = hb

    @pl.when(j >= _NH)
    def _():
        ds_ = dsel_ref[0, 0, :][:, None]
        ws_ = wsel_ref[0, 0, :][:, None]
        tt = lax.broadcasted_iota(jnp.int32, (_HBLK, T), 1)
        E = (0.5 * (tt == ds_).astype(jnp.float32)
             + 0.5 * (tt == ws_).astype(jnp.float32))
        tbl_ref[0] = lax.dot_general(E, h_scr[...],
                                     (((1,), (0,)), ((), ())),
                                     precision=_PREC,
                                     preferred_element_type=jnp.float32)


def _hidden_merge_call(x3, W, b2d, dsel, wsel):
    nj = _NH + _NM
    return pl.pallas_call(
        _hidden_merge_body,
        grid=(B, nj),
        in_specs=[pl.BlockSpec((1, _HBLK, D),
                               lambda b, j: (b, jnp.minimum(j, _NH - 1), 0)),
                  pl.BlockSpec((D, D), lambda b, j: (0, 0)),
                  pl.BlockSpec((1, D), lambda b, j: (0, 0)),
                  pl.BlockSpec((1, 1, _HBLK),
                               lambda b, j: (b, 0, jnp.maximum(j - _NH, 0))),
                  pl.BlockSpec((1, 1, _HBLK),
                               lambda b, j: (b, 0, jnp.maximum(j - _NH, 0)))],
        out_specs=pl.BlockSpec((1, _HBLK, D), lambda b, j: (b, j, 0)),
        out_shape=jax.ShapeDtypeStruct((B, TBL, D), jnp.float32),
        scratch_shapes=[pltpu.VMEM((T, D), jnp.float32)],
    )(x3, W, b2d, dsel, wsel)


_NW = 32
_GCH = 32


def _gather_call(table, gidx):
    nrows = B * T
    per_w = nrows // _NW
    nchk = per_w // _GCH
    mesh = plsc.VectorSubcoreMesh(core_axis_name="c", subcore_axis_name="s")

    @functools.partial(
        pl.kernel,
        out_type=jax.ShapeDtypeStruct((nrows, D), jnp.float32),
        mesh=mesh,
        scratch_types=[pltpu.VMEM((_GCH,), jnp.int32),
                       pltpu.VMEM((_GCH,), jnp.int32),
                       pltpu.VMEM((_GCH, D), jnp.float32),
                       pltpu.VMEM((_GCH, D), jnp.float32),
                       pltpu.SemaphoreType.DMA,
                       pltpu.SemaphoreType.DMA,
                       pltpu.SemaphoreType.DMA,
                       pltpu.SemaphoreType.DMA],
    )
    def k(table_hbm, idx_hbm, out_hbm, iv0, iv1, rv0, rv1, gs0, gs1,
          ws0, ws1):
        wid = lax.axis_index("s") * 2 + lax.axis_index("c")
        base = wid * per_w
        ivs, rvs = (iv0, iv1), (rv0, rv1)
        gss, wss = (gs0, gs1), (ws0, ws1)

        @pl.loop(0, nchk, step=2)
        def _(k0):
            for bb in range(2):
                kk = k0 + bb
                off = base + kk * _GCH

                @pl.when(kk >= 2)
                def _():
                    pltpu.make_async_copy(table_hbm.at[pl.ds(0, _GCH)],
                                          rvs[bb], wss[bb]).wait()

                pltpu.sync_copy(idx_hbm.at[pl.ds(off, _GCH)], ivs[bb])
                pltpu.async_copy(table_hbm.at[ivs[bb]], rvs[bb],
                                 gss[bb]).wait()
                pltpu.async_copy(rvs[bb], out_hbm.at[pl.ds(off, _GCH)],
                                 wss[bb])

        for bb in range(2):
            pltpu.make_async_copy(table_hbm.at[pl.ds(0, _GCH)],
                                  rvs[bb], wss[bb]).wait()

    return k(table, gidx)


def kernel(x, W, b):
    xr = x.reshape(B, S, 2 * D)
    ge, go, dsel, wsel = _scores_select_call(xr)

    table = _hidden_merge_call(x, W, b.reshape(1, D), dsel, wsel)

    g = jnp.stack([ge[:, 0, :], go[:, 0, :]], axis=-1).reshape(B, T)
    g = g + (jnp.arange(B, dtype=jnp.int32) * TBL)[:, None]
    out = _gather_call(table.reshape(B * TBL, D), g.reshape(B * T))
    return out.reshape(B, T, D)
